# baseline (speedup 1.0000x reference)
import functools

import jax
import jax.numpy as jnp
from jax import lax
from jax.experimental import pallas as pl
from jax.experimental.pallas import tpu as pltpu
from jax.experimental.pallas import tpu_sc as plsc

EMBED_DIM = 64
BATCH = 16384


@functools.cache
def _build():
    info = plsc.get_sparse_core_info()
    NC, NS = info.num_cores, info.num_subcores
    NW = NC * NS
    b_per_w = BATCH // NW
    mesh = plsc.VectorSubcoreMesh(core_axis_name="c", subcore_axis_name="s")

    out_t = jax.ShapeDtypeStruct((BATCH, EMBED_DIM), jnp.float32)

    @functools.partial(
        pl.kernel,
        mesh=mesh,
        out_type=[out_t, out_t, out_t],
        compiler_params=pltpu.CompilerParams(use_tc_tiling_on_sc=False),
        scratch_types=[
            pltpu.VMEM((b_per_w,), jnp.int32),
            pltpu.VMEM((b_per_w,), jnp.int32),
            pltpu.VMEM((b_per_w,), jnp.int32),
            pltpu.VMEM((b_per_w, EMBED_DIM), jnp.float32),
            pltpu.VMEM((b_per_w, EMBED_DIM), jnp.float32),
            pltpu.VMEM((b_per_w, EMBED_DIM), jnp.float32),
            pltpu.SemaphoreType.DMA,
        ],
    )
    def lookup(W_u, W_i, W_c, id_u, id_i, id_c, out_u, out_i, out_c,
               idx_u, idx_i, idx_c, rows_u, rows_i, rows_c, sem):
        wid = lax.axis_index("s") * NC + lax.axis_index("c")
        base = wid * b_per_w
        pltpu.sync_copy(id_u.at[pl.ds(base, b_per_w)], idx_u)
        pltpu.sync_copy(id_i.at[pl.ds(base, b_per_w)], idx_i)
        pltpu.sync_copy(id_c.at[pl.ds(base, b_per_w)], idx_c)
        cu = pltpu.async_copy(W_u.at[idx_u], rows_u, sem)
        ci = pltpu.async_copy(W_i.at[idx_i], rows_i, sem)
        cc = pltpu.async_copy(W_c.at[idx_c], rows_c, sem)
        cu.wait()
        pltpu.sync_copy(rows_u, out_u.at[pl.ds(base, b_per_w)])
        ci.wait()
        pltpu.sync_copy(rows_i, out_i.at[pl.ds(base, b_per_w)])
        cc.wait()
        pltpu.sync_copy(rows_c, out_c.at[pl.ds(base, b_per_w)])

    return lookup


def kernel(W_user, W_item, W_category, user_id, item_id, category_id):
    lookup = _build()
    e_user, e_item, e_category = lookup(
        W_user, W_item, W_category,
        user_id.astype(jnp.int32),
        item_id.astype(jnp.int32),
        category_id.astype(jnp.int32),
    )
    return (e_user, e_item, e_category)


# speedup vs baseline: 1.9446x; 1.9446x over previous
import functools

import jax
import jax.numpy as jnp
from jax import lax
from jax.experimental import pallas as pl
from jax.experimental.pallas import tpu as pltpu
from jax.experimental.pallas import tpu_sc as plsc

EMBED_DIM = 64
BATCH = 16384
CHUNK = 64


@functools.cache
def _build():
    info = plsc.get_sparse_core_info()
    NC, NS = info.num_cores, info.num_subcores
    NW = NC * NS
    b_per_w = BATCH // NW
    n_chunks = b_per_w // CHUNK
    mesh = plsc.VectorSubcoreMesh(core_axis_name="c", subcore_axis_name="s")

    out_t = jax.ShapeDtypeStruct((BATCH, EMBED_DIM), jnp.float32)

    @functools.partial(
        pl.kernel,
        mesh=mesh,
        out_type=[out_t, out_t, out_t],
        scratch_types=[
            pltpu.VMEM((b_per_w,), jnp.int32),
            pltpu.VMEM((CHUNK, 8, EMBED_DIM), jnp.float32),
            pltpu.VMEM((CHUNK, EMBED_DIM), jnp.float32),
            pltpu.SemaphoreType.DMA,
        ],
    )
    def lookup(W_u, W_i, W_c, id_u, id_i, id_c, out_u, out_i, out_c,
               idx_v, tiles, obuf, sem):
        wid = lax.axis_index("s") * NC + lax.axis_index("c")
        base = wid * b_per_w

        for W3, ids, out in ((W_u, id_u, out_u),
                             (W_i, id_i, out_i),
                             (W_c, id_c, out_c)):
            pltpu.sync_copy(ids.at[pl.ds(base, b_per_w)], idx_v)

            def chunk_body(c, _, W3=W3, out=out):
                for g in range(CHUNK // 16):
                    v = idx_v[pl.ds(c * CHUNK + g * 16, 16)]
                    for l in range(16):
                        t = lax.shift_right_logical(v[l], 3)
                        pltpu.async_copy(W3.at[t], tiles.at[g * 16 + l], sem)

                def drain(j, _):
                    pltpu.make_async_copy(W3.at[0], tiles.at[0], sem).wait()
                    return _

                lax.fori_loop(0, CHUNK, drain, 0, unroll=8)

                for g in range(CHUNK // 16):
                    rv = lax.bitwise_and(idx_v[pl.ds(c * CHUNK + g * 16, 16)], 7)
                    for l in range(16):
                        r = rv[l]
                        for k in range(EMBED_DIM // 16):
                            obuf[g * 16 + l, pl.ds(16 * k, 16)] = (
                                tiles[g * 16 + l, r, pl.ds(16 * k, 16)]
                            )

                pltpu.sync_copy(obuf, out.at[pl.ds(base + c * CHUNK, CHUNK)])
                return _

            lax.fori_loop(0, n_chunks, chunk_body, 0)

    return lookup


def kernel(W_user, W_item, W_category, user_id, item_id, category_id):
    lookup = _build()
    V_u = W_user.shape[0]
    V_i = W_item.shape[0]
    V_c = W_category.shape[0]
    e_user, e_item, e_category = lookup(
        W_user.reshape(V_u // 8, 8, EMBED_DIM),
        W_item.reshape(V_i // 8, 8, EMBED_DIM),
        W_category.reshape(V_c // 8, 8, EMBED_DIM),
        user_id.astype(jnp.int32),
        item_id.astype(jnp.int32),
        category_id.astype(jnp.int32),
    )
    return (e_user, e_item, e_category)
